# baseline (speedup 1.0000x reference)
import functools

import jax
import jax.numpy as jnp
from jax import lax
from jax.experimental import pallas as pl
from jax.experimental.pallas import tpu as pltpu
from jax.experimental.pallas import tpu_sc as plsc

N = 10000
NP = 10240
E = 320000
D = 128
H = 8
HD = 16
NC = 2
NS = 16
HC = H // NC
DH = D // NC
L = 16
SW = DH + L
NPT = NP // NS
EPT = E // NS
B = 80
NWIN = EPT // B

_f32 = jnp.float32

_GD = lax.GatherDimensionNumbers(
    offset_dims=(), collapsed_slice_dims=(0,), start_index_map=(0,))


def _lane_bcast(v, j):
    idx = jnp.full((L, 1), j, dtype=jnp.int32)
    return lax.gather(v, idx, _GD, slice_sizes=(1,),
                      mode=lax.GatherScatterMode.PROMISE_IN_BOUNDS)


def _prep_body(x_ref, wall_ref, a1_ref, b1_ref, a2_ref, b2_ref,
               sft_ref, f1_ref):
    s = jnp.dot(x_ref[...], wall_ref[...], preferred_element_type=_f32)
    f1 = jnp.dot(s, a1_ref[...], preferred_element_type=_f32) + b1_ref[...]
    f2 = jnp.dot(s, a2_ref[...], preferred_element_type=_f32) + b2_ref[...]
    rep = L // HC
    sft_ref[0] = jnp.concatenate([s[:, :DH]] + [f2[:, :HC]] * rep, axis=1)
    sft_ref[1] = jnp.concatenate([s[:, DH:]] + [f2[:, HC:]] * rep, axis=1)
    f1_ref[0] = jnp.concatenate([f1[:, :HC]] * rep, axis=1)
    f1_ref[1] = jnp.concatenate([f1[:, HC:]] * rep, axis=1)


def _prep(x, wall, a1, b1, a2, b2):
    grid = 10
    rb = NP // grid
    return pl.pallas_call(
        _prep_body,
        grid=(grid,),
        in_specs=[
            pl.BlockSpec((rb, D), lambda i: (i, 0)),
            pl.BlockSpec((D, D), lambda i: (0, 0)),
            pl.BlockSpec((D, H), lambda i: (0, 0)),
            pl.BlockSpec((1, H), lambda i: (0, 0)),
            pl.BlockSpec((D, H), lambda i: (0, 0)),
            pl.BlockSpec((1, H), lambda i: (0, 0)),
        ],
        out_specs=[
            pl.BlockSpec((NC, rb, SW), lambda i: (0, i, 0)),
            pl.BlockSpec((NC, rb, L), lambda i: (0, i, 0)),
        ],
        out_shape=[
            jax.ShapeDtypeStruct((NC, NP, SW), _f32),
            jax.ShapeDtypeStruct((NC, NP, L), _f32),
        ],
    )(x, wall, a1, b1, a2, b2)


def _sc_body(sft_hbm, f1_hbm, row_hbm, col_hbm, z80_hbm,
             acc_out,
             acc_s, sft_s, f1_s,
             rowv0, rowv1, colv0, colv1, srow0, srow1,
             g1v0, g1v1, sfv0, sfv1, isem, gsem, ssem):
    c = lax.axis_index("c")
    s = lax.axis_index("s")
    r0 = s * NPT

    pltpu.sync_copy(z80_hbm.at[pl.ds(r0, NPT)], acc_s.at[pl.ds(r0, NPT)])
    pltpu.sync_copy(sft_hbm.at[c, pl.ds(r0, NPT)], sft_s.at[pl.ds(r0, NPT)])
    pltpu.sync_copy(f1_hbm.at[c, pl.ds(r0, NPT)], f1_s.at[pl.ds(r0, NPT)])
    plsc.subcore_barrier()

    bufs = ((rowv0, colv0, srow0, g1v0, sfv0),
            (rowv1, colv1, srow1, g1v1, sfv1))

    def start_idx(w, b):
        rowv, colv = bufs[b][0], bufs[b][1]
        pltpu.async_copy(row_hbm.at[s, w], rowv, isem.at[b])
        pltpu.async_copy(col_hbm.at[s, w], colv, isem.at[b])

    def wait_idx(w, b):
        rowv, colv = bufs[b][0], bufs[b][1]
        pltpu.make_async_copy(row_hbm.at[s, w], rowv, isem.at[b]).wait()
        pltpu.make_async_copy(col_hbm.at[s, w], colv, isem.at[b]).wait()

    def start_gathers(b):
        rowv, colv, _, g1v, sfv = bufs[b]
        pltpu.async_copy(f1_s.at[rowv], g1v, gsem.at[b])
        pltpu.async_copy(sft_s.at[colv], sfv, gsem.at[b])

    def wait_gathers(b):
        rowv, colv, _, g1v, sfv = bufs[b]
        pltpu.make_async_copy(f1_s.at[rowv], g1v, gsem.at[b]).wait()
        pltpu.make_async_copy(sft_s.at[colv], sfv, gsem.at[b]).wait()

    def start_scatter(b):
        _, _, srow, _, sfv = bufs[b]
        pltpu.async_copy(sfv, acc_s.at[srow], ssem.at[b], add=True)

    def wait_scatter(b):
        _, _, srow, _, sfv = bufs[b]
        pltpu.make_async_copy(sfv, acc_s.at[srow], ssem.at[b]).wait()

    def compute(b):
        g1b, sfb = bufs[b][3], bufs[b][4]

        def edge(e, carry2):
            w = g1b[e] + sfb[e, pl.ds(DH, L)]
            w = jnp.maximum(w * jnp.float32(0.2), w)
            ex = jnp.exp(w)
            for j in range(HC):
                exj = _lane_bcast(ex, j)
                sfb[e, pl.ds(HD * j, HD)] = sfb[e, pl.ds(HD * j, HD)] * exj
            sfb[e, pl.ds(DH, L)] = ex
            return carry2

        lax.fori_loop(0, B, edge, 0, unroll=2)

    start_idx(0, 0)
    wait_idx(0, 0)
    start_gathers(0)

    def body(wo, carry):
        for b in range(2):
            w = 2 * wo + b
            wait_gathers(b)
            rowv, srow = bufs[b][0], bufs[b][2]
            for k in range(B // L):
                srow[pl.ds(k * L, L)] = rowv[pl.ds(k * L, L)]

            @pl.when(w + 1 < NWIN)
            def _():
                start_idx(w + 1, 1 - b)

            @pl.when(w >= 1)
            def _():
                wait_scatter(1 - b)

            @pl.when(w + 1 < NWIN)
            def _():
                wait_idx(w + 1, 1 - b)
                start_gathers(1 - b)

            compute(b)
            start_scatter(b)

        return carry

    lax.fori_loop(0, NWIN // 2, body, 0)
    wait_scatter((NWIN - 1) % 2)
    plsc.subcore_barrier()

    pltpu.sync_copy(acc_s.at[pl.ds(r0, NPT)], acc_out.at[c, pl.ds(r0, NPT)])


_sc_call = functools.partial(
    pl.kernel,
    out_type=jax.ShapeDtypeStruct((NC, NP, SW), _f32),
    mesh=plsc.VectorSubcoreMesh(core_axis_name="c", subcore_axis_name="s"),
    compiler_params=pltpu.CompilerParams(use_tc_tiling_on_sc=False),
    scratch_types=[
        pltpu.VMEM_SHARED((NP, SW), _f32),
        pltpu.VMEM_SHARED((NP, SW), _f32),
        pltpu.VMEM_SHARED((NP, L), _f32),
        pltpu.VMEM((B,), jnp.int32),
        pltpu.VMEM((B,), jnp.int32),
        pltpu.VMEM((B,), jnp.int32),
        pltpu.VMEM((B,), jnp.int32),
        pltpu.VMEM((B,), jnp.int32),
        pltpu.VMEM((B,), jnp.int32),
        pltpu.VMEM((B, L), _f32),
        pltpu.VMEM((B, L), _f32),
        pltpu.VMEM((B, SW), _f32),
        pltpu.VMEM((B, SW), _f32),
        pltpu.SemaphoreType.DMA((2,)),
        pltpu.SemaphoreType.DMA((2,)),
        pltpu.SemaphoreType.DMA((2,)),
    ],
)(_sc_body)


def _fin_body(acc_ref, e0_ref, e1_ref, o_ref):
    n = jnp.concatenate([acc_ref[0, :, :DH], acc_ref[1, :, :DH]], axis=1)
    dex = (jnp.dot(acc_ref[0, :, DH:], e0_ref[...], preferred_element_type=_f32)
           + jnp.dot(acc_ref[1, :, DH:], e1_ref[...], preferred_element_type=_f32))
    h = jnp.where(dex > 0, n / jnp.where(dex > 0, dex, 1.0), 0.0)
    o_ref[...] = jnp.where(h > 0, h, jnp.exp(h) - 1.0)


def _finish(acc, e0, e1):
    grid = 10
    rb = NP // grid
    return pl.pallas_call(
        _fin_body,
        grid=(grid,),
        in_specs=[
            pl.BlockSpec((NC, rb, SW), lambda i: (0, i, 0)),
            pl.BlockSpec((L, D), lambda i: (0, 0)),
            pl.BlockSpec((L, D), lambda i: (0, 0)),
        ],
        out_specs=pl.BlockSpec((rb, D), lambda i: (i, 0)),
        out_shape=jax.ShapeDtypeStruct((NP, D), _f32),
    )(acc, e0, e1)


def kernel(x, edge_index, W, a1_w, a1_b, a2_w, a2_b):
    row = edge_index[0].astype(jnp.int32)
    col = edge_index[1].astype(jnp.int32)

    wall = jnp.transpose(W, (1, 0, 2)).reshape(D, D)
    eye = jnp.eye(H, dtype=_f32)
    a1 = (eye[:, None, :] * a1_w[:, :, 0][:, :, None]).reshape(D, H)
    a2 = (eye[:, None, :] * a2_w[:, :, 0][:, :, None]).reshape(D, H)
    b1 = a1_b[:, 0][None, :]
    b2 = a2_b[:, 0][None, :]

    xp = jnp.pad(x, ((0, NP - N), (0, 0)))
    sft, f1t = _prep(xp, wall, a1, b1, a2, b2)

    z80 = jnp.zeros((NP, SW), _f32)
    row3 = row.reshape(NS, NWIN, B)
    col3 = col.reshape(NS, NWIN, B)
    acc = _sc_call(sft, f1t, row3, col3, z80)

    lane = jnp.arange(L, dtype=jnp.int32)[:, None]
    headcol = (jnp.arange(D, dtype=jnp.int32) // HD)[None, :]
    e0 = ((lane == headcol) & (lane < HC)).astype(_f32)
    e1 = ((lane == (headcol - HC)) & (lane < HC)).astype(_f32)

    return _finish(acc, e0, e1)[:N]


# speedup vs baseline: 1.0055x; 1.0055x over previous
import functools

import jax
import jax.numpy as jnp
from jax import lax
from jax.experimental import pallas as pl
from jax.experimental.pallas import tpu as pltpu
from jax.experimental.pallas import tpu_sc as plsc

N = 10000
NP = 10240
E = 320000
D = 128
H = 8
HD = 16
NC = 2
NS = 16
HC = H // NC
DH = D // NC
L = 16
SW = DH + L
NPT = NP // NS
EPT = E // NS
B = 80
NWIN = EPT // B

_f32 = jnp.float32

_GD = lax.GatherDimensionNumbers(
    offset_dims=(), collapsed_slice_dims=(0,), start_index_map=(0,))


def _lane_bcast(v, j):
    idx = jnp.full((L, 1), j, dtype=jnp.int32)
    return lax.gather(v, idx, _GD, slice_sizes=(1,),
                      mode=lax.GatherScatterMode.PROMISE_IN_BOUNDS)


def _prep_body(x_ref, wall_ref, a1_ref, b1_ref, a2_ref, b2_ref,
               sft_ref, f1_ref):
    s = jnp.dot(x_ref[...], wall_ref[...], preferred_element_type=_f32)
    f1 = jnp.dot(s, a1_ref[...], preferred_element_type=_f32) + b1_ref[...]
    f2 = jnp.dot(s, a2_ref[...], preferred_element_type=_f32) + b2_ref[...]
    rep = L // HC
    sft_ref[0] = jnp.concatenate([s[:, :DH]] + [f2[:, :HC]] * rep, axis=1)
    sft_ref[1] = jnp.concatenate([s[:, DH:]] + [f2[:, HC:]] * rep, axis=1)
    f1_ref[0] = jnp.concatenate([f1[:, :HC]] * rep, axis=1)
    f1_ref[1] = jnp.concatenate([f1[:, HC:]] * rep, axis=1)


def _prep(x, wall, a1, b1, a2, b2):
    grid = 10
    rb = NP // grid
    return pl.pallas_call(
        _prep_body,
        grid=(grid,),
        in_specs=[
            pl.BlockSpec((rb, D), lambda i: (i, 0)),
            pl.BlockSpec((D, D), lambda i: (0, 0)),
            pl.BlockSpec((D, H), lambda i: (0, 0)),
            pl.BlockSpec((1, H), lambda i: (0, 0)),
            pl.BlockSpec((D, H), lambda i: (0, 0)),
            pl.BlockSpec((1, H), lambda i: (0, 0)),
        ],
        out_specs=[
            pl.BlockSpec((NC, rb, SW), lambda i: (0, i, 0)),
            pl.BlockSpec((NC, rb, L), lambda i: (0, i, 0)),
        ],
        out_shape=[
            jax.ShapeDtypeStruct((NC, NP, SW), _f32),
            jax.ShapeDtypeStruct((NC, NP, L), _f32),
        ],
    )(x, wall, a1, b1, a2, b2)


def _sc_body(sft_hbm, f1_hbm, row_hbm, col_hbm, z80_hbm,
             acc_out,
             acc_s, sft_s, f1_s,
             rowv0, rowv1, colv0, colv1, srow0, srow1,
             g1v0, g1v1, sfv0, sfv1, isem, gsem, ssem):
    c = lax.axis_index("c")
    s = lax.axis_index("s")
    r0 = s * NPT

    pltpu.sync_copy(z80_hbm.at[pl.ds(r0, NPT)], acc_s.at[pl.ds(r0, NPT)])
    pltpu.sync_copy(sft_hbm.at[c, pl.ds(r0, NPT)], sft_s.at[pl.ds(r0, NPT)])
    pltpu.sync_copy(f1_hbm.at[c, pl.ds(r0, NPT)], f1_s.at[pl.ds(r0, NPT)])
    plsc.subcore_barrier()

    bufs = ((rowv0, colv0, srow0, g1v0, sfv0),
            (rowv1, colv1, srow1, g1v1, sfv1))

    def start_idx(w, b):
        rowv, colv = bufs[b][0], bufs[b][1]
        pltpu.async_copy(row_hbm.at[s, w], rowv, isem.at[b])
        pltpu.async_copy(col_hbm.at[s, w], colv, isem.at[b])

    def wait_idx(w, b):
        rowv, colv = bufs[b][0], bufs[b][1]
        pltpu.make_async_copy(row_hbm.at[s, w], rowv, isem.at[b]).wait()
        pltpu.make_async_copy(col_hbm.at[s, w], colv, isem.at[b]).wait()

    def start_gathers(b):
        rowv, colv, _, g1v, sfv = bufs[b]
        pltpu.async_copy(f1_s.at[rowv], g1v, gsem.at[b])
        pltpu.async_copy(sft_s.at[colv], sfv, gsem.at[b])

    def wait_gathers(b):
        rowv, colv, _, g1v, sfv = bufs[b]
        pltpu.make_async_copy(f1_s.at[rowv], g1v, gsem.at[b]).wait()
        pltpu.make_async_copy(sft_s.at[colv], sfv, gsem.at[b]).wait()

    def start_scatter(b):
        _, _, srow, _, sfv = bufs[b]
        pltpu.async_copy(sfv, acc_s.at[srow], ssem.at[b], add=True)

    def wait_scatter(b):
        _, _, srow, _, sfv = bufs[b]
        pltpu.make_async_copy(sfv, acc_s.at[srow], ssem.at[b]).wait()

    def compute(b):
        g1b, sfb = bufs[b][3], bufs[b][4]

        def edge(e, carry2):
            w = g1b[e] + sfb[e, pl.ds(DH, L)]
            w = jnp.maximum(w * jnp.float32(0.2), w)
            ex = jnp.exp(w)
            for j in range(HC):
                exj = _lane_bcast(ex, j)
                sfb[e, pl.ds(HD * j, HD)] = sfb[e, pl.ds(HD * j, HD)] * exj
            sfb[e, pl.ds(DH, L)] = ex
            return carry2

        lax.fori_loop(0, B, edge, 0, unroll=2)

    start_idx(0, 0)
    wait_idx(0, 0)
    start_gathers(0)

    def body(wo, carry):
        for b in range(2):
            w = 2 * wo + b
            wait_gathers(b)
            rowv, srow = bufs[b][0], bufs[b][2]
            for k in range(B // L):
                srow[pl.ds(k * L, L)] = rowv[pl.ds(k * L, L)]

            @pl.when(w + 1 < NWIN)
            def _():
                start_idx(w + 1, 1 - b)

            compute(b)
            start_scatter(b)

            @pl.when(w >= 1)
            def _():
                wait_scatter(1 - b)

            @pl.when(w + 1 < NWIN)
            def _():
                wait_idx(w + 1, 1 - b)
                start_gathers(1 - b)

        return carry

    lax.fori_loop(0, NWIN // 2, body, 0)
    wait_scatter((NWIN - 1) % 2)
    plsc.subcore_barrier()

    pltpu.sync_copy(acc_s.at[pl.ds(r0, NPT)], acc_out.at[c, pl.ds(r0, NPT)])


_sc_call = functools.partial(
    pl.kernel,
    out_type=jax.ShapeDtypeStruct((NC, NP, SW), _f32),
    mesh=plsc.VectorSubcoreMesh(core_axis_name="c", subcore_axis_name="s"),
    compiler_params=pltpu.CompilerParams(use_tc_tiling_on_sc=False),
    scratch_types=[
        pltpu.VMEM_SHARED((NP, SW), _f32),
        pltpu.VMEM_SHARED((NP, SW), _f32),
        pltpu.VMEM_SHARED((NP, L), _f32),
        pltpu.VMEM((B,), jnp.int32),
        pltpu.VMEM((B,), jnp.int32),
        pltpu.VMEM((B,), jnp.int32),
        pltpu.VMEM((B,), jnp.int32),
        pltpu.VMEM((B,), jnp.int32),
        pltpu.VMEM((B,), jnp.int32),
        pltpu.VMEM((B, L), _f32),
        pltpu.VMEM((B, L), _f32),
        pltpu.VMEM((B, SW), _f32),
        pltpu.VMEM((B, SW), _f32),
        pltpu.SemaphoreType.DMA((2,)),
        pltpu.SemaphoreType.DMA((2,)),
        pltpu.SemaphoreType.DMA((2,)),
    ],
)(_sc_body)


def _fin_body(acc_ref, e0_ref, e1_ref, o_ref):
    n = jnp.concatenate([acc_ref[0, :, :DH], acc_ref[1, :, :DH]], axis=1)
    dex = (jnp.dot(acc_ref[0, :, DH:], e0_ref[...], preferred_element_type=_f32)
           + jnp.dot(acc_ref[1, :, DH:], e1_ref[...], preferred_element_type=_f32))
    h = jnp.where(dex > 0, n / jnp.where(dex > 0, dex, 1.0), 0.0)
    o_ref[...] = jnp.where(h > 0, h, jnp.exp(h) - 1.0)


def _finish(acc, e0, e1):
    grid = 10
    rb = NP // grid
    return pl.pallas_call(
        _fin_body,
        grid=(grid,),
        in_specs=[
            pl.BlockSpec((NC, rb, SW), lambda i: (0, i, 0)),
            pl.BlockSpec((L, D), lambda i: (0, 0)),
            pl.BlockSpec((L, D), lambda i: (0, 0)),
        ],
        out_specs=pl.BlockSpec((rb, D), lambda i: (i, 0)),
        out_shape=jax.ShapeDtypeStruct((NP, D), _f32),
    )(acc, e0, e1)


def kernel(x, edge_index, W, a1_w, a1_b, a2_w, a2_b):
    row = edge_index[0].astype(jnp.int32)
    col = edge_index[1].astype(jnp.int32)

    wall = jnp.transpose(W, (1, 0, 2)).reshape(D, D)
    eye = jnp.eye(H, dtype=_f32)
    a1 = (eye[:, None, :] * a1_w[:, :, 0][:, :, None]).reshape(D, H)
    a2 = (eye[:, None, :] * a2_w[:, :, 0][:, :, None]).reshape(D, H)
    b1 = a1_b[:, 0][None, :]
    b2 = a2_b[:, 0][None, :]

    xp = jnp.pad(x, ((0, NP - N), (0, 0)))
    sft, f1t = _prep(xp, wall, a1, b1, a2, b2)

    z80 = jnp.zeros((NP, SW), _f32)
    row3 = row.reshape(NS, NWIN, B)
    col3 = col.reshape(NS, NWIN, B)
    acc = _sc_call(sft, f1t, row3, col3, z80)

    lane = jnp.arange(L, dtype=jnp.int32)[:, None]
    headcol = (jnp.arange(D, dtype=jnp.int32) // HD)[None, :]
    e0 = ((lane == headcol) & (lane < HC)).astype(_f32)
    e1 = ((lane == (headcol - HC)) & (lane < HC)).astype(_f32)

    return _finish(acc, e0, e1)[:N]
